# baseline (speedup 1.0000x reference)
import functools

import jax
import jax.numpy as jnp
from jax import lax
from jax.experimental import pallas as pl
from jax.experimental.pallas import tpu as pltpu
from jax.experimental.pallas import tpu_sc as plsc

NC = 19
B = 4
NPIX = 512 * 512
NWORK = 32
WPB = NWORK // B
WPIX = NPIX // WPB
CP = 4096
NCHUNK = WPIX // CP
NGRP = CP // 16
COLS = 48
ACC_W = NC * COLS
DEN = 1e-08


def _sc_stats_kernel(logits_hbm, gt_hbm, out_hbm, lp_v, gt_v, acc_v, tot_v):
    wid = lax.axis_index("s") * 2 + lax.axis_index("c")
    batch = wid // WPB
    sub = wid % WPB
    zeros16 = jnp.zeros((16,), jnp.float32)
    ones16 = jnp.ones((16,), jnp.float32)
    lane = lax.iota(jnp.int32, 16)

    def _zero(j, _):
        acc_v[pl.ds(j * 16, 16)] = zeros16
        return 0

    lax.fori_loop(0, ACC_W * 16 // 16, _zero, 0)

    def _chunk(j, _):
        off = sub * WPIX + j * CP
        for c in range(NC):
            pltpu.sync_copy(
                logits_hbm.at[pl.ds((batch * NC + c) * NPIX + off, CP)],
                lp_v.at[pl.ds(c * CP, CP)])
        pltpu.sync_copy(gt_hbm.at[pl.ds(batch * NPIX + off, CP)], gt_v)

        def _group(i, _):
            g = gt_v[pl.ds(i * 16, 16)]
            v0 = lp_v[pl.ds(i * 16, 16)]
            m = v0
            q = v0 * v0
            gl = v0
            vs = [v0]
            for c in range(1, NC):
                vc = lp_v[pl.ds(c * CP + i * 16, 16)]
                vs.append(vc)
                m = jnp.maximum(m, vc)
                q = q + vc * vc
                gl = jnp.where(g == c, vc, gl)
            tp = gl >= m
            base = lane * ACC_W + g * COLS
            for c in range(NC):
                plsc.addupdate_scatter(acc_v, [base + c], vs[c])
            plsc.addupdate_scatter(acc_v, [base + NC], q)
            plsc.addupdate_scatter(acc_v, [base + NC + 1], ones16)
            for c in range(NC):
                plsc.addupdate_scatter(acc_v, [base + 24 + c], vs[c], mask=tp)
            plsc.addupdate_scatter(acc_v, [base + 43], ones16, mask=tp)
            return 0

        lax.fori_loop(0, NGRP, _group, 0)
        return 0

    lax.fori_loop(0, NCHUNK, _chunk, 0)

    def _red(j, _):
        t = acc_v[pl.ds(j * 16, 16)]
        for l in range(1, 16):
            t = t + acc_v[pl.ds(l * ACC_W + j * 16, 16)]
        tot_v[pl.ds(j * 16, 16)] = t
        return 0

    lax.fori_loop(0, ACC_W // 16, _red, 0)
    pltpu.sync_copy(tot_v, out_hbm.at[pl.ds(wid * ACC_W, ACC_W)])


@functools.partial(
    pl.kernel,
    out_type=jax.ShapeDtypeStruct((NWORK * ACC_W,), jnp.float32),
    mesh=plsc.VectorSubcoreMesh(core_axis_name="c", subcore_axis_name="s"),
    compiler_params=pltpu.CompilerParams(use_tc_tiling_on_sc=False,
                                         needs_layout_passes=False),
    scratch_types=[
        pltpu.VMEM((NC * CP,), jnp.float32),
        pltpu.VMEM((CP,), jnp.int32),
        pltpu.VMEM((16 * ACC_W,), jnp.float32),
        pltpu.VMEM((ACC_W,), jnp.float32),
    ],
)
def _sc_stats(logits_hbm, gt_hbm, out_hbm, lp_v, gt_v, acc_v, tot_v):
    _sc_stats_kernel(logits_hbm, gt_hbm, out_hbm, lp_v, gt_v, acc_v, tot_v)


def _combine_body(p_ref, out_ref):
    r = p_ref[0:NC, :]
    for w in range(1, NWORK):
        r = r + p_ref[w * NC:(w + 1) * NC, :]
    s_mat = r[:, :NC]
    q_col = r[:, NC:NC + 1]
    n_col = r[:, NC + 1:NC + 2]
    stp_mat = r[:, 24:24 + NC]
    ntp_col = r[:, 43:44]

    has_tp = ntp_col > 0.0
    mav = jnp.where(has_tp, stp_mat / jnp.maximum(ntp_col, 1.0), 0.0)
    cross = jnp.sum(mav * s_mat, axis=1, keepdims=True)
    mavsq = jnp.sum(mav * mav, axis=1, keepdims=True)
    sq = q_col - 2.0 * cross + n_col * mavsq
    term = sq / (jnp.maximum(n_col, 1.0) * float(NC)) / DEN

    labels = lax.broadcasted_iota(jnp.int32, (NC, 1), 0)
    present = n_col > 0.0
    max_present = jnp.max(jnp.where(present, labels, -1))
    include = present & (labels != max_present) & has_tp
    out_ref[...] = jnp.sum(jnp.where(include, term, 0.0),
                           axis=0, keepdims=True)


@jax.jit
def _ow_loss_sc(logits, sem_gt):
    partials = _sc_stats(logits.reshape(B * NC * NPIX), sem_gt.reshape(B * NPIX))
    p2 = partials.reshape(NWORK * NC, COLS)
    out = pl.pallas_call(
        _combine_body,
        out_shape=jax.ShapeDtypeStruct((1, 1), jnp.float32),
    )(p2)
    return out[0, 0]


def kernel(logits, sem_gt, is_train):
    loss = _ow_loss_sc(logits, sem_gt)
    return jnp.where(is_train != 0, loss, jnp.array(0.0, jnp.float32))


# speedup vs baseline: 1.3471x; 1.3471x over previous
import functools

import jax
import jax.numpy as jnp
from jax import lax
from jax.experimental import pallas as pl
from jax.experimental.pallas import tpu as pltpu
from jax.experimental.pallas import tpu_sc as plsc

NC = 19
B = 4
NPIX = 512 * 512
NWORK = 32
WPB = NWORK // B
WPIX = NPIX // WPB
CP = 4096
NCHUNK = WPIX // CP
NGRP = CP // 16
COLS = 24
NROW = 2 * NC
ACC_W = NROW * COLS
DEN = 1e-08


def _sc_stats_kernel(logits_hbm, gt_hbm, out_hbm, lp_v, gt_v, acc_v, tot_v):
    wid = lax.axis_index("s") * 2 + lax.axis_index("c")
    batch = wid // WPB
    sub = wid % WPB
    zeros16 = jnp.zeros((16,), jnp.float32)
    ones16 = jnp.ones((16,), jnp.float32)
    lane = lax.iota(jnp.int32, 16)

    def _zero(j, _):
        acc_v[pl.ds(j * 16, 16)] = zeros16
        return 0

    lax.fori_loop(0, ACC_W * 16 // 16, _zero, 0)

    def _chunk(j, _):
        off = sub * WPIX + j * CP
        for c in range(NC):
            pltpu.sync_copy(
                logits_hbm.at[pl.ds((batch * NC + c) * NPIX + off, CP)],
                lp_v.at[pl.ds(c * CP, CP)])
        pltpu.sync_copy(gt_hbm.at[pl.ds(batch * NPIX + off, CP)], gt_v)

        def _one_group(i):
            g = gt_v[pl.ds(i * 16, 16)]
            v0 = lp_v[pl.ds(i * 16, 16)]
            m = v0
            q = v0 * v0
            gl = v0
            vs = [v0]
            for c in range(1, NC):
                vc = lp_v[pl.ds(c * CP + i * 16, 16)]
                vs.append(vc)
                m = jnp.maximum(m, vc)
                q = q + vc * vc
                gl = jnp.where(g == c, vc, gl)
            row = g + jnp.where(gl >= m, 0, NC)
            base = lane * ACC_W + row * COLS
            for c in range(NC):
                plsc.addupdate_scatter(acc_v, [base + c], vs[c])
            plsc.addupdate_scatter(acc_v, [base + NC], q)
            plsc.addupdate_scatter(acc_v, [base + NC + 1], ones16)

        def _group(i, _):
            _one_group(2 * i)
            _one_group(2 * i + 1)
            return 0

        lax.fori_loop(0, NGRP // 2, _group, 0)
        return 0

    lax.fori_loop(0, NCHUNK, _chunk, 0)

    def _red(j, _):
        t = acc_v[pl.ds(j * 16, 16)]
        for l in range(1, 16):
            t = t + acc_v[pl.ds(l * ACC_W + j * 16, 16)]
        tot_v[pl.ds(j * 16, 16)] = t
        return 0

    lax.fori_loop(0, ACC_W // 16, _red, 0)
    pltpu.sync_copy(tot_v, out_hbm.at[pl.ds(wid * ACC_W, ACC_W)])


@functools.partial(
    pl.kernel,
    out_type=jax.ShapeDtypeStruct((NWORK * ACC_W,), jnp.float32),
    mesh=plsc.VectorSubcoreMesh(core_axis_name="c", subcore_axis_name="s"),
    compiler_params=pltpu.CompilerParams(use_tc_tiling_on_sc=False,
                                         needs_layout_passes=False),
    scratch_types=[
        pltpu.VMEM((NC * CP,), jnp.float32),
        pltpu.VMEM((CP,), jnp.int32),
        pltpu.VMEM((16 * ACC_W,), jnp.float32),
        pltpu.VMEM((ACC_W,), jnp.float32),
    ],
)
def _sc_stats(logits_hbm, gt_hbm, out_hbm, lp_v, gt_v, acc_v, tot_v):
    _sc_stats_kernel(logits_hbm, gt_hbm, out_hbm, lp_v, gt_v, acc_v, tot_v)


def _combine_body(p_ref, out_ref):
    r = p_ref[0:NROW, :]
    for w in range(1, NWORK):
        r = r + p_ref[w * NROW:(w + 1) * NROW, :]
    tpart = r[:NC]
    npart = r[NC:]
    s_mat = tpart[:, :NC] + npart[:, :NC]
    q_col = tpart[:, NC:NC + 1] + npart[:, NC:NC + 1]
    n_col = tpart[:, NC + 1:NC + 2] + npart[:, NC + 1:NC + 2]
    stp_mat = tpart[:, :NC]
    ntp_col = tpart[:, NC + 1:NC + 2]

    has_tp = ntp_col > 0.0
    mav = jnp.where(has_tp, stp_mat / jnp.maximum(ntp_col, 1.0), 0.0)
    cross = jnp.sum(mav * s_mat, axis=1, keepdims=True)
    mavsq = jnp.sum(mav * mav, axis=1, keepdims=True)
    sq = q_col - 2.0 * cross + n_col * mavsq
    term = sq / (jnp.maximum(n_col, 1.0) * float(NC)) / DEN

    labels = lax.broadcasted_iota(jnp.int32, (NC, 1), 0)
    present = n_col > 0.0
    max_present = jnp.max(jnp.where(present, labels, -1))
    include = present & (labels != max_present) & has_tp
    out_ref[...] = jnp.sum(jnp.where(include, term, 0.0),
                           axis=0, keepdims=True)


@jax.jit
def _ow_loss_sc(logits, sem_gt):
    partials = _sc_stats(logits.reshape(B * NC * NPIX), sem_gt.reshape(B * NPIX))
    p2 = partials.reshape(NWORK * NROW, COLS)
    out = pl.pallas_call(
        _combine_body,
        out_shape=jax.ShapeDtypeStruct((1, 1), jnp.float32),
    )(p2)
    return out[0, 0]


def kernel(logits, sem_gt, is_train):
    loss = _ow_loss_sc(logits, sem_gt)
    return jnp.where(is_train != 0, loss, jnp.array(0.0, jnp.float32))


# speedup vs baseline: 2.3260x; 1.7266x over previous
import functools

import jax
import jax.numpy as jnp
from jax import lax
from jax.experimental import pallas as pl
from jax.experimental.pallas import tpu as pltpu
from jax.experimental.pallas import tpu_sc as plsc

NC = 19
B = 4
NPIX = 512 * 512
DEN = 1e-08

SC_PIX = 65536
NWORK = 32
WPB = NWORK // B
WPIX = SC_PIX // WPB
CP = 4096
NCHUNK = WPIX // CP
NGRP = CP // 16
COLS = 24
NROW = 2 * NC
ACC_W = NROW * COLS

PBLK = 32768
TC_PIX = NPIX - SC_PIX
NBLK = TC_PIX // PBLK
SKIP = SC_PIX // PBLK


def _sc_stats_kernel(logits_hbm, gt_hbm, out_hbm, lp_v, gt_v, acc_v, tot_v):
    wid = lax.axis_index("s") * 2 + lax.axis_index("c")
    batch = wid // WPB
    sub = wid % WPB
    zeros16 = jnp.zeros((16,), jnp.float32)
    ones16 = jnp.ones((16,), jnp.float32)
    lane = lax.iota(jnp.int32, 16)

    def _zero(j, _):
        acc_v[pl.ds(j * 16, 16)] = zeros16
        return 0

    lax.fori_loop(0, ACC_W * 16 // 16, _zero, 0)

    def _chunk(j, _):
        off = sub * WPIX + j * CP
        for c in range(NC):
            pltpu.sync_copy(
                logits_hbm.at[pl.ds((batch * NC + c) * NPIX + off, CP)],
                lp_v.at[pl.ds(c * CP, CP)])
        pltpu.sync_copy(gt_hbm.at[pl.ds(batch * NPIX + off, CP)], gt_v)

        def _one_group(i):
            g = gt_v[pl.ds(i * 16, 16)]
            v0 = lp_v[pl.ds(i * 16, 16)]
            m = v0
            q = v0 * v0
            gl = v0
            vs = [v0]
            for c in range(1, NC):
                vc = lp_v[pl.ds(c * CP + i * 16, 16)]
                vs.append(vc)
                m = jnp.maximum(m, vc)
                q = q + vc * vc
                gl = jnp.where(g == c, vc, gl)
            row = g + jnp.where(gl >= m, 0, NC)
            base = lane * ACC_W + row * COLS
            for c in range(NC):
                plsc.addupdate_scatter(acc_v, [base + c], vs[c])
            plsc.addupdate_scatter(acc_v, [base + NC], q)
            plsc.addupdate_scatter(acc_v, [base + NC + 1], ones16)

        def _group(i, _):
            _one_group(2 * i)
            _one_group(2 * i + 1)
            return 0

        lax.fori_loop(0, NGRP // 2, _group, 0)
        return 0

    lax.fori_loop(0, NCHUNK, _chunk, 0)

    def _red(j, _):
        t = acc_v[pl.ds(j * 16, 16)]
        for l in range(1, 16):
            t = t + acc_v[pl.ds(l * ACC_W + j * 16, 16)]
        tot_v[pl.ds(j * 16, 16)] = t
        return 0

    lax.fori_loop(0, ACC_W // 16, _red, 0)
    pltpu.sync_copy(tot_v, out_hbm.at[pl.ds(wid * ACC_W, ACC_W)])


@functools.partial(
    pl.kernel,
    out_type=jax.ShapeDtypeStruct((NWORK * ACC_W,), jnp.float32),
    mesh=plsc.VectorSubcoreMesh(core_axis_name="c", subcore_axis_name="s"),
    compiler_params=pltpu.CompilerParams(use_tc_tiling_on_sc=False,
                                         needs_layout_passes=False),
    scratch_types=[
        pltpu.VMEM((NC * CP,), jnp.float32),
        pltpu.VMEM((CP,), jnp.int32),
        pltpu.VMEM((16 * ACC_W,), jnp.float32),
        pltpu.VMEM((ACC_W,), jnp.float32),
    ],
)
def _sc_stats(logits_hbm, gt_hbm, out_hbm, lp_v, gt_v, acc_v, tot_v):
    _sc_stats_kernel(logits_hbm, gt_hbm, out_hbm, lp_v, gt_v, acc_v, tot_v)


def _tc_body(logits_ref, gt_ref, out_ref):
    step = pl.program_id(0) * NBLK + pl.program_id(1)

    lp = logits_ref[0]
    gt = gt_ref[0]

    cls = lax.broadcasted_iota(jnp.int32, (NC, PBLK), 0)
    m = jnp.max(lp, axis=0, keepdims=True)
    q = jnp.sum(lp * lp, axis=0, keepdims=True)

    onehot = jnp.where(gt == cls, 1.0, 0.0)
    tp = jnp.where(lp >= m, onehot, 0.0)

    aug = jnp.concatenate(
        [lp, q, jnp.ones((1, PBLK), jnp.float32)], axis=0
    ).astype(jnp.bfloat16)
    oh_bf = onehot.astype(jnp.bfloat16)
    tp_bf = tp.astype(jnp.bfloat16)

    dims = (((1,), (1,)), ((), ()))
    s_all = lax.dot_general(oh_bf, aug, dims,
                            preferred_element_type=jnp.float32)
    s_tp = lax.dot_general(tp_bf, aug, dims,
                           preferred_element_type=jnp.float32)

    @pl.when(step == 0)
    def _init():
        out_ref[:NC] = s_all
        out_ref[NC:] = s_tp

    @pl.when(step != 0)
    def _accum():
        out_ref[:NC] += s_all
        out_ref[NC:] += s_tp


def _combine_body(sc_ref, tc_ref, out_ref):
    r = sc_ref[0:NROW, :]
    for w in range(1, NWORK):
        r = r + sc_ref[w * NROW:(w + 1) * NROW, :]
    tc_all = tc_ref[:NC, :]
    tc_tp = tc_ref[NC:, :]

    sc_tp = r[:NC, :NC + 2]
    sc_np = r[NC:, :NC + 2]

    allst = sc_tp + sc_np + tc_all
    tpst = sc_tp + tc_tp

    s_mat = allst[:, :NC]
    q_col = allst[:, NC:NC + 1]
    n_col = allst[:, NC + 1:NC + 2]
    stp_mat = tpst[:, :NC]
    ntp_col = tpst[:, NC + 1:NC + 2]

    has_tp = ntp_col > 0.0
    mav = jnp.where(has_tp, stp_mat / jnp.maximum(ntp_col, 1.0), 0.0)
    cross = jnp.sum(mav * s_mat, axis=1, keepdims=True)
    mavsq = jnp.sum(mav * mav, axis=1, keepdims=True)
    sq = q_col - 2.0 * cross + n_col * mavsq
    term = sq / (jnp.maximum(n_col, 1.0) * float(NC)) / DEN

    labels = lax.broadcasted_iota(jnp.int32, (NC, 1), 0)
    present = n_col > 0.0
    max_present = jnp.max(jnp.where(present, labels, -1))
    include = present & (labels != max_present) & has_tp
    out_ref[...] = jnp.sum(jnp.where(include, term, 0.0),
                           axis=0, keepdims=True)


@jax.jit
def _ow_loss(logits, sem_gt):
    flat_lp = logits.reshape(B * NC * NPIX)
    flat_gt = sem_gt.reshape(B * NPIX)
    sc_partials = _sc_stats(flat_lp, flat_gt)

    logits3 = logits.reshape(B, NC, NPIX)
    gt3 = sem_gt.reshape(B * (NPIX // PBLK), 1, PBLK)
    tc_stats = pl.pallas_call(
        _tc_body,
        grid=(B, NBLK),
        in_specs=[
            pl.BlockSpec((1, NC, PBLK), lambda b, j: (b, 0, j + SKIP)),
            pl.BlockSpec((1, 1, PBLK),
                         lambda b, j: (b * (NPIX // PBLK) + j + SKIP, 0, 0)),
        ],
        out_specs=pl.BlockSpec((2 * NC, NC + 2), lambda b, j: (0, 0)),
        out_shape=jax.ShapeDtypeStruct((2 * NC, NC + 2), jnp.float32),
    )(logits3, gt3)

    out = pl.pallas_call(
        _combine_body,
        out_shape=jax.ShapeDtypeStruct((1, 1), jnp.float32),
    )(sc_partials.reshape(NWORK * NROW, COLS), tc_stats)
    return out[0, 0]


def kernel(logits, sem_gt, is_train):
    loss = _ow_loss(logits, sem_gt)
    return jnp.where(is_train != 0, loss, jnp.array(0.0, jnp.float32))


# speedup vs baseline: 2.5900x; 1.1135x over previous
import functools

import jax
import jax.numpy as jnp
from jax import lax
from jax.experimental import pallas as pl
from jax.experimental.pallas import tpu as pltpu
from jax.experimental.pallas import tpu_sc as plsc

NC = 19
B = 4
NPIX = 512 * 512
DEN = 1e-08

SC_PIX = 65536
NWORK = 32
WPB = NWORK // B
WPIX = SC_PIX // WPB
CP = 4096
NCHUNK = WPIX // CP
NGRP = CP // 16
COLS = 24
NROW = 2 * NC
ACC_W = NROW * COLS

PBLK = 32768
TC_PIX = NPIX - SC_PIX
NBLK = TC_PIX // PBLK
SKIP = SC_PIX // PBLK
PR = PBLK // 128
NROWS = NPIX // 128


def _sc_stats_kernel(logits_hbm, gt_hbm, out_hbm, lp_v, gt_v, acc_v, tot_v):
    wid = lax.axis_index("s") * 2 + lax.axis_index("c")
    batch = wid // WPB
    sub = wid % WPB
    zeros16 = jnp.zeros((16,), jnp.float32)
    ones16 = jnp.ones((16,), jnp.float32)
    lane = lax.iota(jnp.int32, 16)

    def _zero(j, _):
        acc_v[pl.ds(j * 16, 16)] = zeros16
        return 0

    lax.fori_loop(0, ACC_W * 16 // 16, _zero, 0)

    def _chunk(j, _):
        off = sub * WPIX + j * CP
        for c in range(NC):
            pltpu.sync_copy(
                logits_hbm.at[pl.ds((batch * NC + c) * NPIX + off, CP)],
                lp_v.at[pl.ds(c * CP, CP)])
        pltpu.sync_copy(gt_hbm.at[pl.ds(batch * NPIX + off, CP)], gt_v)

        def _one_group(i):
            g = gt_v[pl.ds(i * 16, 16)]
            v0 = lp_v[pl.ds(i * 16, 16)]
            m = v0
            q = v0 * v0
            gl = v0
            vs = [v0]
            for c in range(1, NC):
                vc = lp_v[pl.ds(c * CP + i * 16, 16)]
                vs.append(vc)
                m = jnp.maximum(m, vc)
                q = q + vc * vc
                gl = jnp.where(g == c, vc, gl)
            row = g + jnp.where(gl >= m, 0, NC)
            base = lane * ACC_W + row * COLS
            for c in range(NC):
                plsc.addupdate_scatter(acc_v, [base + c], vs[c])
            plsc.addupdate_scatter(acc_v, [base + NC], q)
            plsc.addupdate_scatter(acc_v, [base + NC + 1], ones16)

        def _group(i, _):
            _one_group(2 * i)
            _one_group(2 * i + 1)
            return 0

        lax.fori_loop(0, NGRP // 2, _group, 0)
        return 0

    lax.fori_loop(0, NCHUNK, _chunk, 0)

    def _red(j, _):
        t = acc_v[pl.ds(j * 16, 16)]
        for l in range(1, 16):
            t = t + acc_v[pl.ds(l * ACC_W + j * 16, 16)]
        tot_v[pl.ds(j * 16, 16)] = t
        return 0

    lax.fori_loop(0, ACC_W // 16, _red, 0)
    pltpu.sync_copy(tot_v, out_hbm.at[pl.ds(wid * ACC_W, ACC_W)])


@functools.partial(
    pl.kernel,
    out_type=jax.ShapeDtypeStruct((NWORK * ACC_W,), jnp.float32),
    mesh=plsc.VectorSubcoreMesh(core_axis_name="c", subcore_axis_name="s"),
    compiler_params=pltpu.CompilerParams(use_tc_tiling_on_sc=False,
                                         needs_layout_passes=False),
    scratch_types=[
        pltpu.VMEM((NC * CP,), jnp.float32),
        pltpu.VMEM((CP,), jnp.int32),
        pltpu.VMEM((16 * ACC_W,), jnp.float32),
        pltpu.VMEM((ACC_W,), jnp.float32),
    ],
)
def _sc_stats(logits_hbm, gt_hbm, out_hbm, lp_v, gt_v, acc_v, tot_v):
    _sc_stats_kernel(logits_hbm, gt_hbm, out_hbm, lp_v, gt_v, acc_v, tot_v)


def _tc_body(logits_ref, gt_ref, out_ref):
    step = pl.program_id(0) * NBLK + pl.program_id(1)

    lp3 = logits_ref[...]
    gt3 = gt_ref[...]
    lp = lp3.reshape(NC, PBLK)
    gt = gt3.reshape(1, PBLK)

    cls = lax.broadcasted_iota(jnp.int32, (NC, PBLK), 0)
    m = jnp.max(lp, axis=0, keepdims=True)
    q = jnp.sum(lp * lp, axis=0, keepdims=True)

    onehot = jnp.where(gt == cls, 1.0, 0.0)
    tp = jnp.where(lp >= m, onehot, 0.0)

    aug = jnp.concatenate(
        [lp, q, jnp.ones((1, PBLK), jnp.float32)], axis=0
    ).astype(jnp.bfloat16)
    oh_bf = onehot.astype(jnp.bfloat16)
    tp_bf = tp.astype(jnp.bfloat16)

    dims = (((1,), (1,)), ((), ()))
    s_all = lax.dot_general(oh_bf, aug, dims,
                            preferred_element_type=jnp.float32)
    s_tp = lax.dot_general(tp_bf, aug, dims,
                           preferred_element_type=jnp.float32)

    @pl.when(step == 0)
    def _init():
        out_ref[:NC] = s_all
        out_ref[NC:] = s_tp

    @pl.when(step != 0)
    def _accum():
        out_ref[:NC] += s_all
        out_ref[NC:] += s_tp


def _combine_body(sc_ref, tc_ref, out_ref):
    r = sc_ref[0:NROW, :]
    for w in range(1, NWORK):
        r = r + sc_ref[w * NROW:(w + 1) * NROW, :]
    tc_all = tc_ref[:NC, :]
    tc_tp = tc_ref[NC:, :]

    sc_tp = r[:NC, :NC + 2]
    sc_np = r[NC:, :NC + 2]

    allst = sc_tp + sc_np + tc_all
    tpst = sc_tp + tc_tp

    s_mat = allst[:, :NC]
    q_col = allst[:, NC:NC + 1]
    n_col = allst[:, NC + 1:NC + 2]
    stp_mat = tpst[:, :NC]
    ntp_col = tpst[:, NC + 1:NC + 2]

    has_tp = ntp_col > 0.0
    mav = jnp.where(has_tp, stp_mat / jnp.maximum(ntp_col, 1.0), 0.0)
    cross = jnp.sum(mav * s_mat, axis=1, keepdims=True)
    mavsq = jnp.sum(mav * mav, axis=1, keepdims=True)
    sq = q_col - 2.0 * cross + n_col * mavsq
    term = sq / (jnp.maximum(n_col, 1.0) * float(NC)) / DEN

    labels = lax.broadcasted_iota(jnp.int32, (NC, 1), 0)
    present = n_col > 0.0
    max_present = jnp.max(jnp.where(present, labels, -1))
    include = present & (labels != max_present) & has_tp
    out_ref[...] = jnp.sum(jnp.where(include, term, 0.0),
                           axis=0, keepdims=True)


@jax.jit
def _ow_loss(logits, sem_gt):
    flat_lp = logits.reshape(B * NC * NPIX)
    flat_gt = sem_gt.reshape(B * NPIX)
    sc_partials = _sc_stats(flat_lp, flat_gt)

    logits3 = flat_lp.reshape(B * NC, NROWS, 128)
    gt3 = flat_gt.reshape(B, NROWS, 128)
    tc_stats = pl.pallas_call(
        _tc_body,
        grid=(B, NBLK),
        in_specs=[
            pl.BlockSpec((NC, PR, 128), lambda b, j: (b, j + SKIP, 0)),
            pl.BlockSpec((1, PR, 128), lambda b, j: (b, j + SKIP, 0)),
        ],
        out_specs=pl.BlockSpec((2 * NC, NC + 2), lambda b, j: (0, 0)),
        out_shape=jax.ShapeDtypeStruct((2 * NC, NC + 2), jnp.float32),
    )(logits3, gt3)

    out = pl.pallas_call(
        _combine_body,
        out_shape=jax.ShapeDtypeStruct((1, 1), jnp.float32),
    )(sc_partials.reshape(NWORK * NROW, COLS), tc_stats)
    return out[0, 0]


def kernel(logits, sem_gt, is_train):
    loss = _ow_loss(logits, sem_gt)
    return jnp.where(is_train != 0, loss, jnp.array(0.0, jnp.float32))


# speedup vs baseline: 3.7992x; 1.4669x over previous
import functools

import jax
import jax.numpy as jnp
from jax import lax
from jax.experimental import pallas as pl
from jax.experimental.pallas import tpu as pltpu
from jax.experimental.pallas import tpu_sc as plsc

NC = 19
B = 4
NPIX = 512 * 512
DEN = 1e-08

SC_ROWS = 128
SC_PIX = SC_ROWS * 512
NWORK = 32
WPB = NWORK // B
WPIX = SC_PIX // WPB
CP = 4096
NCHUNK = WPIX // CP
NGRP = CP // 16
COLS = 24
NROW = 2 * NC
ACC_W = NROW * COLS

PBLK = 32768
TC_PIX = NPIX - SC_PIX
NBLK = TC_PIX // PBLK
RH = PBLK // 512
SKIP = SC_ROWS // RH


def _sc_stats_kernel(logits_hbm, gt_hbm, out_hbm, lp_v, gt_v, acc_v, tot_v):
    wid = lax.axis_index("s") * 2 + lax.axis_index("c")
    batch = wid // WPB
    sub = wid % WPB
    zeros16 = jnp.zeros((16,), jnp.float32)
    ones16 = jnp.ones((16,), jnp.float32)
    lane = lax.iota(jnp.int32, 16)

    def _zero(j, _):
        acc_v[pl.ds(j * 16, 16)] = zeros16
        return 0

    lax.fori_loop(0, ACC_W * 16 // 16, _zero, 0)

    def _chunk(j, _):
        off = sub * WPIX + j * CP
        for c in range(NC):
            pltpu.sync_copy(
                logits_hbm.at[pl.ds((batch * NC + c) * SC_PIX + off, CP)],
                lp_v.at[pl.ds(c * CP, CP)])
        pltpu.sync_copy(gt_hbm.at[pl.ds(batch * SC_PIX + off, CP)], gt_v)

        def _one_group(i):
            g = gt_v[pl.ds(i * 16, 16)]
            v0 = lp_v[pl.ds(i * 16, 16)]
            m = v0
            q = v0 * v0
            gl = v0
            vs = [v0]
            for c in range(1, NC):
                vc = lp_v[pl.ds(c * CP + i * 16, 16)]
                vs.append(vc)
                m = jnp.maximum(m, vc)
                q = q + vc * vc
                gl = jnp.where(g == c, vc, gl)
            row = g + jnp.where(gl >= m, 0, NC)
            base = lane * ACC_W + row * COLS
            for c in range(NC):
                plsc.addupdate_scatter(acc_v, [base + c], vs[c])
            plsc.addupdate_scatter(acc_v, [base + NC], q)
            plsc.addupdate_scatter(acc_v, [base + NC + 1], ones16)

        def _group(i, _):
            _one_group(2 * i)
            _one_group(2 * i + 1)
            return 0

        lax.fori_loop(0, NGRP // 2, _group, 0)
        return 0

    lax.fori_loop(0, NCHUNK, _chunk, 0)

    def _red(j, _):
        t = acc_v[pl.ds(j * 16, 16)]
        for l in range(1, 16):
            t = t + acc_v[pl.ds(l * ACC_W + j * 16, 16)]
        tot_v[pl.ds(j * 16, 16)] = t
        return 0

    lax.fori_loop(0, ACC_W // 16, _red, 0)
    pltpu.sync_copy(tot_v, out_hbm.at[pl.ds(wid * ACC_W, ACC_W)])


@functools.partial(
    pl.kernel,
    out_type=jax.ShapeDtypeStruct((NWORK * ACC_W,), jnp.float32),
    mesh=plsc.VectorSubcoreMesh(core_axis_name="c", subcore_axis_name="s"),
    compiler_params=pltpu.CompilerParams(use_tc_tiling_on_sc=False,
                                         needs_layout_passes=False),
    scratch_types=[
        pltpu.VMEM((NC * CP,), jnp.float32),
        pltpu.VMEM((CP,), jnp.int32),
        pltpu.VMEM((16 * ACC_W,), jnp.float32),
        pltpu.VMEM((ACC_W,), jnp.float32),
    ],
)
def _sc_stats(logits_hbm, gt_hbm, out_hbm, lp_v, gt_v, acc_v, tot_v):
    _sc_stats_kernel(logits_hbm, gt_hbm, out_hbm, lp_v, gt_v, acc_v, tot_v)


def _tc_body(logits_ref, gt_ref, out_ref):
    step = pl.program_id(0) * NBLK + pl.program_id(1)

    lp = logits_ref[0].reshape(NC, PBLK)
    gt = gt_ref[0].reshape(1, PBLK)

    cls = lax.broadcasted_iota(jnp.int32, (NC, PBLK), 0)
    m = jnp.max(lp, axis=0, keepdims=True)
    q = jnp.sum(lp * lp, axis=0, keepdims=True)

    onehot = jnp.where(gt == cls, 1.0, 0.0)
    tp = jnp.where(lp >= m, onehot, 0.0)

    aug = jnp.concatenate(
        [lp, q, jnp.ones((1, PBLK), jnp.float32)], axis=0
    ).astype(jnp.bfloat16)
    oh_bf = onehot.astype(jnp.bfloat16)
    tp_bf = tp.astype(jnp.bfloat16)

    dims = (((1,), (1,)), ((), ()))
    s_all = lax.dot_general(oh_bf, aug, dims,
                            preferred_element_type=jnp.float32)
    s_tp = lax.dot_general(tp_bf, aug, dims,
                           preferred_element_type=jnp.float32)

    @pl.when(step == 0)
    def _init():
        out_ref[:NC] = s_all
        out_ref[NC:] = s_tp

    @pl.when(step != 0)
    def _accum():
        out_ref[:NC] += s_all
        out_ref[NC:] += s_tp


def _combine_body(sc_ref, tc_ref, out_ref):
    r = sc_ref[0:NROW, :]
    for w in range(1, NWORK):
        r = r + sc_ref[w * NROW:(w + 1) * NROW, :]
    tc_all = tc_ref[:NC, :]
    tc_tp = tc_ref[NC:, :]

    sc_tp = r[:NC, :NC + 2]
    sc_np = r[NC:, :NC + 2]

    allst = sc_tp + sc_np + tc_all
    tpst = sc_tp + tc_tp

    s_mat = allst[:, :NC]
    q_col = allst[:, NC:NC + 1]
    n_col = allst[:, NC + 1:NC + 2]
    stp_mat = tpst[:, :NC]
    ntp_col = tpst[:, NC + 1:NC + 2]

    has_tp = ntp_col > 0.0
    mav = jnp.where(has_tp, stp_mat / jnp.maximum(ntp_col, 1.0), 0.0)
    cross = jnp.sum(mav * s_mat, axis=1, keepdims=True)
    mavsq = jnp.sum(mav * mav, axis=1, keepdims=True)
    sq = q_col - 2.0 * cross + n_col * mavsq
    term = sq / (jnp.maximum(n_col, 1.0) * float(NC)) / DEN

    labels = lax.broadcasted_iota(jnp.int32, (NC, 1), 0)
    present = n_col > 0.0
    max_present = jnp.max(jnp.where(present, labels, -1))
    include = present & (labels != max_present) & has_tp
    out_ref[...] = jnp.sum(jnp.where(include, term, 0.0),
                           axis=0, keepdims=True)


@jax.jit
def _ow_loss(logits, sem_gt):
    sc_lp = logits[:, :, :SC_ROWS, :].reshape(B * NC * SC_PIX)
    sc_gt = sem_gt[:, :SC_ROWS, :].reshape(B * SC_PIX)
    sc_partials = _sc_stats(sc_lp, sc_gt)

    tc_stats = pl.pallas_call(
        _tc_body,
        grid=(B, NBLK),
        in_specs=[
            pl.BlockSpec((1, NC, RH, 512), lambda b, j: (b, 0, j + SKIP, 0)),
            pl.BlockSpec((1, RH, 512), lambda b, j: (b, j + SKIP, 0)),
        ],
        out_specs=pl.BlockSpec((2 * NC, NC + 2), lambda b, j: (0, 0)),
        out_shape=jax.ShapeDtypeStruct((2 * NC, NC + 2), jnp.float32),
    )(logits, sem_gt)

    out = pl.pallas_call(
        _combine_body,
        out_shape=jax.ShapeDtypeStruct((1, 1), jnp.float32),
    )(sc_partials.reshape(NWORK * NROW, COLS), tc_stats)
    return out[0, 0]


def kernel(logits, sem_gt, is_train):
    loss = _ow_loss(logits, sem_gt)
    return jnp.where(is_train != 0, loss, jnp.array(0.0, jnp.float32))


# speedup vs baseline: 3.8089x; 1.0025x over previous
import functools

import jax
import jax.numpy as jnp
from jax import lax
from jax.experimental import pallas as pl
from jax.experimental.pallas import tpu as pltpu
from jax.experimental.pallas import tpu_sc as plsc

NC = 19
B = 4
NPIX = 512 * 512
DEN = 1e-08

SC_ROWS = 128
SC_PIX = SC_ROWS * 512
NWORK = 32
WPB = NWORK // B
WPIX = SC_PIX // WPB
CP = 4096
NCHUNK = WPIX // CP
NGRP = CP // 16
COLS = 24
NROW = 2 * NC
ACC_W = NROW * COLS

PBLK = 16384
TC_PIX = NPIX - SC_PIX
NBLK = TC_PIX // PBLK
RH = PBLK // 512
SKIP = SC_ROWS // RH


def _sc_stats_kernel(logits_hbm, gt_hbm, out_hbm, lp_v, gt_v, acc_v, tot_v):
    wid = lax.axis_index("s") * 2 + lax.axis_index("c")
    batch = wid // WPB
    sub = wid % WPB
    zeros16 = jnp.zeros((16,), jnp.float32)
    ones16 = jnp.ones((16,), jnp.float32)
    lane = lax.iota(jnp.int32, 16)

    def _zero(j, _):
        acc_v[pl.ds(j * 16, 16)] = zeros16
        return 0

    lax.fori_loop(0, ACC_W * 16 // 16, _zero, 0)

    def _chunk(j, _):
        off = sub * WPIX + j * CP
        for c in range(NC):
            pltpu.sync_copy(
                logits_hbm.at[pl.ds((batch * NC + c) * SC_PIX + off, CP)],
                lp_v.at[pl.ds(c * CP, CP)])
        pltpu.sync_copy(gt_hbm.at[pl.ds(batch * SC_PIX + off, CP)], gt_v)

        def _one_group(i):
            g = gt_v[pl.ds(i * 16, 16)]
            v0 = lp_v[pl.ds(i * 16, 16)]
            m = v0
            q = v0 * v0
            gl = v0
            vs = [v0]
            for c in range(1, NC):
                vc = lp_v[pl.ds(c * CP + i * 16, 16)]
                vs.append(vc)
                m = jnp.maximum(m, vc)
                q = q + vc * vc
                gl = jnp.where(g == c, vc, gl)
            row = g + jnp.where(gl >= m, 0, NC)
            base = lane * ACC_W + row * COLS
            for c in range(NC):
                plsc.addupdate_scatter(acc_v, [base + c], vs[c])
            plsc.addupdate_scatter(acc_v, [base + NC], q)
            plsc.addupdate_scatter(acc_v, [base + NC + 1], ones16)

        def _group(i, _):
            _one_group(2 * i)
            _one_group(2 * i + 1)
            return 0

        lax.fori_loop(0, NGRP // 2, _group, 0)
        return 0

    lax.fori_loop(0, NCHUNK, _chunk, 0)

    def _red(j, _):
        t = acc_v[pl.ds(j * 16, 16)]
        for l in range(1, 16):
            t = t + acc_v[pl.ds(l * ACC_W + j * 16, 16)]
        tot_v[pl.ds(j * 16, 16)] = t
        return 0

    lax.fori_loop(0, ACC_W // 16, _red, 0)
    pltpu.sync_copy(tot_v, out_hbm.at[pl.ds(wid * ACC_W, ACC_W)])


@functools.partial(
    pl.kernel,
    out_type=jax.ShapeDtypeStruct((NWORK * ACC_W,), jnp.float32),
    mesh=plsc.VectorSubcoreMesh(core_axis_name="c", subcore_axis_name="s"),
    compiler_params=pltpu.CompilerParams(use_tc_tiling_on_sc=False,
                                         needs_layout_passes=False),
    scratch_types=[
        pltpu.VMEM((NC * CP,), jnp.float32),
        pltpu.VMEM((CP,), jnp.int32),
        pltpu.VMEM((16 * ACC_W,), jnp.float32),
        pltpu.VMEM((ACC_W,), jnp.float32),
    ],
)
def _sc_stats(logits_hbm, gt_hbm, out_hbm, lp_v, gt_v, acc_v, tot_v):
    _sc_stats_kernel(logits_hbm, gt_hbm, out_hbm, lp_v, gt_v, acc_v, tot_v)


def _tc_body(logits_ref, gt_ref, out_ref):
    step = pl.program_id(0) * NBLK + pl.program_id(1)

    lp = logits_ref[0].reshape(NC, PBLK)
    gt = gt_ref[0].reshape(1, PBLK)

    cls = lax.broadcasted_iota(jnp.int32, (NC, PBLK), 0)
    m = jnp.max(lp, axis=0, keepdims=True)
    q = jnp.sum(lp * lp, axis=0, keepdims=True)

    onehot = jnp.where(gt == cls, 1.0, 0.0)
    tp = jnp.where(lp >= m, onehot, 0.0)

    aug = jnp.concatenate(
        [lp, q, jnp.ones((1, PBLK), jnp.float32)], axis=0
    ).astype(jnp.bfloat16)
    oh_bf = onehot.astype(jnp.bfloat16)
    tp_bf = tp.astype(jnp.bfloat16)

    dims = (((1,), (1,)), ((), ()))
    s_all = lax.dot_general(oh_bf, aug, dims,
                            preferred_element_type=jnp.float32)
    s_tp = lax.dot_general(tp_bf, aug, dims,
                           preferred_element_type=jnp.float32)

    @pl.when(step == 0)
    def _init():
        out_ref[:NC] = s_all
        out_ref[NC:] = s_tp

    @pl.when(step != 0)
    def _accum():
        out_ref[:NC] += s_all
        out_ref[NC:] += s_tp


def _combine_body(sc_ref, tc_ref, out_ref):
    r = sc_ref[0:NROW, :]
    for w in range(1, NWORK):
        r = r + sc_ref[w * NROW:(w + 1) * NROW, :]
    tc_all = tc_ref[:NC, :]
    tc_tp = tc_ref[NC:, :]

    sc_tp = r[:NC, :NC + 2]
    sc_np = r[NC:, :NC + 2]

    allst = sc_tp + sc_np + tc_all
    tpst = sc_tp + tc_tp

    s_mat = allst[:, :NC]
    q_col = allst[:, NC:NC + 1]
    n_col = allst[:, NC + 1:NC + 2]
    stp_mat = tpst[:, :NC]
    ntp_col = tpst[:, NC + 1:NC + 2]

    has_tp = ntp_col > 0.0
    mav = jnp.where(has_tp, stp_mat / jnp.maximum(ntp_col, 1.0), 0.0)
    cross = jnp.sum(mav * s_mat, axis=1, keepdims=True)
    mavsq = jnp.sum(mav * mav, axis=1, keepdims=True)
    sq = q_col - 2.0 * cross + n_col * mavsq
    term = sq / (jnp.maximum(n_col, 1.0) * float(NC)) / DEN

    labels = lax.broadcasted_iota(jnp.int32, (NC, 1), 0)
    present = n_col > 0.0
    max_present = jnp.max(jnp.where(present, labels, -1))
    include = present & (labels != max_present) & has_tp
    out_ref[...] = jnp.sum(jnp.where(include, term, 0.0),
                           axis=0, keepdims=True)


@jax.jit
def _ow_loss(logits, sem_gt):
    sc_lp = logits[:, :, :SC_ROWS, :].reshape(B * NC * SC_PIX)
    sc_gt = sem_gt[:, :SC_ROWS, :].reshape(B * SC_PIX)
    sc_partials = _sc_stats(sc_lp, sc_gt)

    tc_stats = pl.pallas_call(
        _tc_body,
        grid=(B, NBLK),
        in_specs=[
            pl.BlockSpec((1, NC, RH, 512), lambda b, j: (b, 0, j + SKIP, 0)),
            pl.BlockSpec((1, RH, 512), lambda b, j: (b, j + SKIP, 0)),
        ],
        out_specs=pl.BlockSpec((2 * NC, NC + 2), lambda b, j: (0, 0)),
        out_shape=jax.ShapeDtypeStruct((2 * NC, NC + 2), jnp.float32),
    )(logits, sem_gt)

    out = pl.pallas_call(
        _combine_body,
        out_shape=jax.ShapeDtypeStruct((1, 1), jnp.float32),
    )(sc_partials.reshape(NWORK * NROW, COLS), tc_stats)
    return out[0, 0]


def kernel(logits, sem_gt, is_train):
    loss = _ow_loss(logits, sem_gt)
    return jnp.where(is_train != 0, loss, jnp.array(0.0, jnp.float32))
